# baseline (speedup 1.0000x reference)
import jax
import jax.numpy as jnp
from jax import lax
from jax.experimental import pallas as pl
from jax.experimental.pallas import tpu as pltpu
from jax.experimental.pallas import tpu_sc as plsc

B = 16384
D = 64
DP = 128
NW = 32
BW = B // NW
PASS_ROWS = 256
NCHUNK = 2
CHUNK = 128
PASS_GROUPS = PASS_ROWS // 16
MAGIC = 0x5F3759DF


def _sqrt_pos(x):
    y = lax.bitcast_convert_type(
        jnp.int32(MAGIC) - (lax.bitcast_convert_type(x, jnp.int32) >> 1),
        jnp.float32)
    half = x * 0.5
    for _ in range(3):
        y = y * (1.5 - half * y * y)
    return x * y


def _body(uidx_hbm, iidx_hbm, table_hbm, out_hbm,
          uidx_v, iidx_v, uslot_v, islot_v, a_v, b_v, out_v, sem):
    wid = lax.axis_index("s") * 2 + lax.axis_index("c")

    lane = lax.iota(jnp.int32, 16)
    zero = jnp.zeros((16,), jnp.float32)

    for p in range(2):
        pltpu.sync_copy(uidx_hbm.at[wid, pl.ds(p * NCHUNK, NCHUNK)], uidx_v)
        pltpu.sync_copy(iidx_hbm.at[wid, pl.ds(p * NCHUNK, NCHUNK)], iidx_v)
        for j in range(NCHUNK):
            for k in range(CHUNK // 16):
                s = pl.ds(k * 16, 16)
                uslot_v[j, s] = uidx_v[j, s] >> 1
                islot_v[j, s] = iidx_v[j, s] >> 1

        copies = []
        for j in range(NCHUNK):
            rows = pl.ds(j * CHUNK, CHUNK)
            copies.append(pltpu.async_copy(
                table_hbm.at[uslot_v.at[j]], a_v.at[rows], sem))
            copies.append(pltpu.async_copy(
                table_hbm.at[islot_v.at[j]], b_v.at[rows], sem))
        for c in copies:
            c.wait()

        for g in range(PASS_GROUPS):
            row_ids = lane + (g * 16)
            s16 = pl.ds((g % 8) * 16, 16)
            acol = (uidx_v[g // 8, s16] & 1) * D
            bcol = (iidx_v[g // 8, s16] & 1) * D

            def dstep(i, carry):
                sn, sa, sb = carry
                d0 = i * 4
                for u in range(4):
                    av = plsc.load_gather(a_v, [row_ids, acol + (d0 + u)])
                    bv = plsc.load_gather(b_v, [row_ids, bcol + (d0 + u)])
                    sn = sn + av * bv
                    sa = sa + av * av
                    sb = sb + bv * bv
                return sn, sa, sb

            sn, sa, sb = lax.fori_loop(0, D // 4, dstep, (zero, zero, zero))

            na = jnp.maximum(_sqrt_pos(sa), 1e-8)
            nb = jnp.maximum(_sqrt_pos(sb), 1e-8)
            slot = p * PASS_GROUPS + g
            out_v[slot // 8, pl.ds((slot % 8) * 16, 16)] = sn / (na * nb)

    pltpu.sync_copy(out_v, out_hbm.at[pl.ds(wid * 4, 4)])


def kernel(user_idx, item_idx, user_table, item_table):
    del item_table
    uidx = user_idx.astype(jnp.int32).reshape(NW, 2 * NCHUNK, CHUNK)
    iidx = item_idx.astype(jnp.int32).reshape(NW, 2 * NCHUNK, CHUNK)
    t2 = user_table.reshape(-1, DP)

    f = pl.kernel(
        _body,
        out_type=jax.ShapeDtypeStruct((NW * 4, 128), jnp.float32),
        mesh=plsc.VectorSubcoreMesh(core_axis_name="c", subcore_axis_name="s"),
        compiler_params=pltpu.CompilerParams(
            needs_layout_passes=False, use_tc_tiling_on_sc=True),
        scratch_types=[
            pltpu.VMEM((NCHUNK, CHUNK), jnp.int32),
            pltpu.VMEM((NCHUNK, CHUNK), jnp.int32),
            pltpu.VMEM((NCHUNK, CHUNK), jnp.int32),
            pltpu.VMEM((NCHUNK, CHUNK), jnp.int32),
            pltpu.VMEM((PASS_ROWS, DP), jnp.float32),
            pltpu.VMEM((PASS_ROWS, DP), jnp.float32),
            pltpu.VMEM((4, 128), jnp.float32),
            pltpu.SemaphoreType.DMA,
        ],
    )
    out = f(uidx, iidx, t2)
    return out.reshape(B, 1)


# speedup vs baseline: 1.1009x; 1.1009x over previous
import jax
import jax.numpy as jnp
from jax import lax
from jax.experimental import pallas as pl
from jax.experimental.pallas import tpu as pltpu
from jax.experimental.pallas import tpu_sc as plsc

B = 16384
D = 64
DP = 128
NW = 32
BW = B // NW
PASS_ROWS = 256
NCHUNK = 2
CHUNK = 128
PASS_GROUPS = PASS_ROWS // 16
MAGIC = 0x5F3759DF


def _sqrt_pos(x):
    y = lax.bitcast_convert_type(
        jnp.int32(MAGIC) - (lax.bitcast_convert_type(x, jnp.int32) >> 1),
        jnp.float32)
    half = x * 0.5
    for _ in range(3):
        y = y * (1.5 - half * y * y)
    return x * y


def _body(uidx_hbm, iidx_hbm, table_hbm, out_hbm,
          uidx_v, iidx_v, a_v, b_v, out_v, sem):
    wid = lax.axis_index("s") * 2 + lax.axis_index("c")

    lane = lax.iota(jnp.int32, 16)
    zero = jnp.zeros((16,), jnp.float32)

    for p in range(2):
        pltpu.sync_copy(uidx_hbm.at[wid, pl.ds(p * NCHUNK, NCHUNK)], uidx_v)
        pltpu.sync_copy(iidx_hbm.at[wid, pl.ds(p * NCHUNK, NCHUNK)], iidx_v)

        copies = []
        for j in range(NCHUNK):
            rows = pl.ds(j * CHUNK, CHUNK)
            copies.append(pltpu.async_copy(
                table_hbm.at[uidx_v.at[j]], a_v.at[rows], sem))
            copies.append(pltpu.async_copy(
                table_hbm.at[iidx_v.at[j]], b_v.at[rows], sem))
        for c in copies:
            c.wait()

        for g in range(PASS_GROUPS):
            row_ids = lane + (g * 16)

            def dstep(i, carry):
                sn, sa, sb = carry
                d0 = i * 4
                for u in range(4):
                    col = jnp.full((16,), d0 + u, jnp.int32)
                    av = plsc.load_gather(a_v, [row_ids, col])
                    bv = plsc.load_gather(b_v, [row_ids, col])
                    sn = sn + av * bv
                    sa = sa + av * av
                    sb = sb + bv * bv
                return sn, sa, sb

            sn, sa, sb = lax.fori_loop(0, D // 4, dstep, (zero, zero, zero))

            na = jnp.maximum(_sqrt_pos(sa), 1e-8)
            nb = jnp.maximum(_sqrt_pos(sb), 1e-8)
            slot = p * PASS_GROUPS + g
            out_v[slot // 8, pl.ds((slot % 8) * 16, 16)] = sn / (na * nb)

    pltpu.sync_copy(out_v, out_hbm.at[pl.ds(wid * 4, 4)])


def kernel(user_idx, item_idx, user_table, item_table):
    del item_table
    uidx = user_idx.astype(jnp.int32).reshape(NW, 2 * NCHUNK, CHUNK)
    iidx = item_idx.astype(jnp.int32).reshape(NW, 2 * NCHUNK, CHUNK)
    tp = jnp.pad(user_table, ((0, 0), (0, DP - D)))

    f = pl.kernel(
        _body,
        out_type=jax.ShapeDtypeStruct((NW * 4, 128), jnp.float32),
        mesh=plsc.VectorSubcoreMesh(core_axis_name="c", subcore_axis_name="s"),
        compiler_params=pltpu.CompilerParams(
            needs_layout_passes=False, use_tc_tiling_on_sc=True),
        scratch_types=[
            pltpu.VMEM((NCHUNK, CHUNK), jnp.int32),
            pltpu.VMEM((NCHUNK, CHUNK), jnp.int32),
            pltpu.VMEM((PASS_ROWS, DP), jnp.float32),
            pltpu.VMEM((PASS_ROWS, DP), jnp.float32),
            pltpu.VMEM((4, 128), jnp.float32),
            pltpu.SemaphoreType.DMA,
        ],
    )
    out = f(uidx, iidx, tp)
    return out.reshape(B, 1)


# speedup vs baseline: 1.6441x; 1.4935x over previous
import jax
import jax.numpy as jnp
from jax import lax
from jax.experimental import pallas as pl
from jax.experimental.pallas import tpu as pltpu
from jax.experimental.pallas import tpu_sc as plsc

B = 16384
D = 64
DP = 128
NW = 32
BW = B // NW
PASS_ROWS = 256
NCHUNK = 2
CHUNK = 128
PASS_GROUPS = PASS_ROWS // 16
MAGIC = 0x5F3759DF


def _sqrt_pos(x):
    y = lax.bitcast_convert_type(
        jnp.int32(MAGIC) - (lax.bitcast_convert_type(x, jnp.int32) >> 1),
        jnp.float32)
    half = x * 0.5
    for _ in range(3):
        y = y * (1.5 - half * y * y)
    return x * y


def _body(uidx_hbm, iidx_hbm, table_hbm, out_hbm,
          uidx_v, iidx_v, a_v, b_v, out_v, sem):
    wid = lax.axis_index("s") * 2 + lax.axis_index("c")

    lane = lax.iota(jnp.int32, 16)
    zero = jnp.zeros((16,), jnp.float32)

    for p in range(2):
        pltpu.sync_copy(uidx_hbm.at[wid, pl.ds(p * NCHUNK, NCHUNK)], uidx_v)
        pltpu.sync_copy(iidx_hbm.at[wid, pl.ds(p * NCHUNK, NCHUNK)], iidx_v)

        def fetch(g, _):
            base = g * 16
            va = uidx_v[base // CHUNK, pl.ds(base % CHUNK, 16)]
            vb = iidx_v[base // CHUNK, pl.ds(base % CHUNK, 16)]
            for u in range(16):
                pltpu.async_copy(table_hbm.at[pl.ds(va[u], 1)],
                                 a_v.at[pl.ds(base + u, 1)], sem)
                pltpu.async_copy(table_hbm.at[pl.ds(vb[u], 1)],
                                 b_v.at[pl.ds(base + u, 1)], sem)
            return 0

        lax.fori_loop(0, PASS_ROWS // 16, fetch, 0)
        pltpu.make_async_copy(
            table_hbm.at[pl.ds(0, PASS_ROWS)], a_v, sem).wait()
        pltpu.make_async_copy(
            table_hbm.at[pl.ds(0, PASS_ROWS)], b_v, sem).wait()

        for g in range(PASS_GROUPS):
            row_ids = lane + (g * 16)

            def dstep(i, carry):
                sn, sa, sb = carry
                d0 = i * 4
                for u in range(4):
                    col = jnp.full((16,), d0 + u, jnp.int32)
                    av = plsc.load_gather(a_v, [row_ids, col])
                    bv = plsc.load_gather(b_v, [row_ids, col])
                    sn = sn + av * bv
                    sa = sa + av * av
                    sb = sb + bv * bv
                return sn, sa, sb

            sn, sa, sb = lax.fori_loop(0, D // 4, dstep, (zero, zero, zero))

            na = jnp.maximum(_sqrt_pos(sa), 1e-8)
            nb = jnp.maximum(_sqrt_pos(sb), 1e-8)
            slot = p * PASS_GROUPS + g
            out_v[slot // 8, pl.ds((slot % 8) * 16, 16)] = sn / (na * nb)

    pltpu.sync_copy(out_v, out_hbm.at[pl.ds(wid * 4, 4)])


def kernel(user_idx, item_idx, user_table, item_table):
    del item_table
    uidx = user_idx.astype(jnp.int32).reshape(NW, 2 * NCHUNK, CHUNK)
    iidx = item_idx.astype(jnp.int32).reshape(NW, 2 * NCHUNK, CHUNK)
    tp = user_table

    f = pl.kernel(
        _body,
        out_type=jax.ShapeDtypeStruct((NW * 4, 128), jnp.float32),
        mesh=plsc.VectorSubcoreMesh(core_axis_name="c", subcore_axis_name="s"),
        compiler_params=pltpu.CompilerParams(
            needs_layout_passes=False, use_tc_tiling_on_sc=True),
        scratch_types=[
            pltpu.VMEM((NCHUNK, CHUNK), jnp.int32),
            pltpu.VMEM((NCHUNK, CHUNK), jnp.int32),
            pltpu.VMEM((PASS_ROWS, D), jnp.float32),
            pltpu.VMEM((PASS_ROWS, D), jnp.float32),
            pltpu.VMEM((4, 128), jnp.float32),
            pltpu.SemaphoreType.DMA,
        ],
    )
    out = f(uidx, iidx, tp)
    return out.reshape(B, 1)


# speedup vs baseline: 1.6479x; 1.0023x over previous
import jax
import jax.numpy as jnp
from jax import lax
from jax.experimental import pallas as pl
from jax.experimental.pallas import tpu as pltpu
from jax.experimental.pallas import tpu_sc as plsc

B = 16384
D = 64
DP = 128
NW = 32
BW = B // NW
PASS_ROWS = 256
NCHUNK = 2
CHUNK = 128
PASS_GROUPS = PASS_ROWS // 16
MAGIC = 0x5F3759DF


def _sqrt_pos(x):
    y = lax.bitcast_convert_type(
        jnp.int32(MAGIC) - (lax.bitcast_convert_type(x, jnp.int32) >> 1),
        jnp.float32)
    half = x * 0.5
    for _ in range(3):
        y = y * (1.5 - half * y * y)
    return x * y


def _body(uidx_hbm, iidx_hbm, table_hbm, out_hbm,
          uidx_v, iidx_v, a_v, b_v, out_v, sem):
    wid = lax.axis_index("s") * 2 + lax.axis_index("c")

    lane = lax.iota(jnp.int32, 16)
    zero = jnp.zeros((16,), jnp.float32)

    for p in range(2):
        pltpu.sync_copy(uidx_hbm.at[wid, pl.ds(p * NCHUNK, NCHUNK)], uidx_v)
        pltpu.sync_copy(iidx_hbm.at[wid, pl.ds(p * NCHUNK, NCHUNK)], iidx_v)

        def fetch(g, _):
            base = g * 16
            va = uidx_v[base // CHUNK, pl.ds(base % CHUNK, 16)]
            vb = iidx_v[base // CHUNK, pl.ds(base % CHUNK, 16)]
            for u in range(16):
                pltpu.async_copy(table_hbm.at[pl.ds(va[u], 1)],
                                 a_v.at[pl.ds(base + u, 1)], sem)
                pltpu.async_copy(table_hbm.at[pl.ds(vb[u], 1)],
                                 b_v.at[pl.ds(base + u, 1)], sem)
            return 0

        lax.fori_loop(0, PASS_ROWS // 16, fetch, 0)
        pltpu.make_async_copy(
            table_hbm.at[pl.ds(0, PASS_ROWS)], a_v, sem).wait()
        pltpu.make_async_copy(
            table_hbm.at[pl.ds(0, PASS_ROWS)], b_v, sem).wait()

        for g in range(PASS_GROUPS):
            row_ids = lane + (g * 16)

            def dstep(i, carry):
                sn, sa, sb = carry
                d0 = i * 4
                for u in range(4):
                    col = jnp.full((16,), d0 + u, jnp.int32)
                    av = plsc.load_gather(a_v, [row_ids, col])
                    bv = plsc.load_gather(b_v, [row_ids, col])
                    sn = sn + av * bv
                    sa = sa + av * av
                    sb = sb + bv * bv
                return sn, sa, sb

            sn, sa, sb = lax.fori_loop(0, D // 4, dstep, (zero, zero, zero))

            na = jnp.maximum(_sqrt_pos(sa), 1e-8)
            nb = jnp.maximum(_sqrt_pos(sb), 1e-8)
            slot = p * PASS_GROUPS + g
            out_v[slot // 8, pl.ds((slot % 8) * 16, 16)] = sn / (na * nb)

    pltpu.sync_copy(out_v, out_hbm.at[pl.ds(wid * 4, 4)])


def kernel(user_idx, item_idx, user_table, item_table):
    del item_table
    uidx = user_idx.astype(jnp.int32).reshape(NW, 2 * NCHUNK, CHUNK)
    iidx = item_idx.astype(jnp.int32).reshape(NW, 2 * NCHUNK, CHUNK)
    tp = lax.optimization_barrier(user_table)

    f = pl.kernel(
        _body,
        out_type=jax.ShapeDtypeStruct((NW * 4, 128), jnp.float32),
        mesh=plsc.VectorSubcoreMesh(core_axis_name="c", subcore_axis_name="s"),
        compiler_params=pltpu.CompilerParams(
            needs_layout_passes=False, use_tc_tiling_on_sc=True),
        scratch_types=[
            pltpu.VMEM((NCHUNK, CHUNK), jnp.int32),
            pltpu.VMEM((NCHUNK, CHUNK), jnp.int32),
            pltpu.VMEM((PASS_ROWS, D), jnp.float32),
            pltpu.VMEM((PASS_ROWS, D), jnp.float32),
            pltpu.VMEM((4, 128), jnp.float32),
            pltpu.SemaphoreType.DMA,
        ],
    )
    out = f(uidx, iidx, tp)
    return out.reshape(B, 1)
